# baseline (speedup 1.0000x reference)
import functools

import jax
import jax.numpy as jnp
from jax import lax
from jax.experimental import pallas as pl
from jax.experimental.pallas import tpu as pltpu
from jax.experimental.pallas import tpu_sc as plsc

K = 100000
D = 16
B = 16384

_NW = 32
_BPW = B // _NW
_SUB = 8
_G1 = B // _SUB

_TC_ROWS = 2048
_TC_G = _TC_ROWS // _SUB


def _make_gather_kernel():
  mesh = plsc.VectorSubcoreMesh(core_axis_name="c", subcore_axis_name="s")

  @functools.partial(
      pl.kernel,
      mesh=mesh,
      compiler_params=pltpu.CompilerParams(use_tc_tiling_on_sc=False),
      out_type=jax.ShapeDtypeStruct((_G1, _SUB, 128), jnp.float32),
      scratch_types=[
          pltpu.VMEM((_BPW,), jnp.int32),
          pltpu.VMEM((_BPW, D), jnp.float32),
          pltpu.VMEM((_BPW // _SUB, _SUB, 128), jnp.float32),
          pltpu.SemaphoreType.DMA,
      ],
  )
  def k(a_hbm, m_hbm, g_hbm, idx_v, rows_v, pack_v, sem):
    wid = lax.axis_index("s") * 2 + lax.axis_index("c")
    base = wid * _BPW
    pltpu.sync_copy(a_hbm.at[pl.ds(base, _BPW)], idx_v)
    pltpu.async_copy(m_hbm.at[idx_v], rows_v, sem).wait()

    def body(t, _):
      for j in range(_SUB):
        pack_v[t, j, pl.ds(0, D)] = rows_v[t * _SUB + j, :]
      return _

    lax.fori_loop(0, _BPW // _SUB, body, 0)
    pltpu.sync_copy(pack_v, g_hbm.at[pl.ds(wid * (_BPW // _SUB), _BPW // _SUB)])

  return k


_gather = _make_gather_kernel()


def _dot_sigmoid_body(x_ref, g_ref, o_ref):
  g = g_ref[...][:, :, :D].reshape(_TC_ROWS, D)
  p = x_ref[...] * g
  z = jnp.sum(p, axis=1)
  o_ref[...] = 1.0 / (1.0 + jnp.exp(-z))


_dot_sigmoid = pl.pallas_call(
    _dot_sigmoid_body,
    grid=(B // _TC_ROWS,),
    in_specs=[
        pl.BlockSpec((_TC_ROWS, D), lambda i: (i, 0)),
        pl.BlockSpec((_TC_G, _SUB, 128), lambda i: (i, 0, 0)),
    ],
    out_specs=pl.BlockSpec((_TC_ROWS,), lambda i: (i,)),
    out_shape=jax.ShapeDtypeStruct((B,), jnp.float32),
)


@jax.jit
def kernel(X, A, m):
  g3 = _gather(A.astype(jnp.int32), m)
  return _dot_sigmoid(X, g3)


# speedup vs baseline: 1.3478x; 1.3478x over previous
import functools

import jax
import jax.numpy as jnp
from jax import lax
from jax.experimental import pallas as pl
from jax.experimental.pallas import tpu as pltpu
from jax.experimental.pallas import tpu_sc as plsc

K = 100000
D = 16
B = 16384

_NW = 32
_BPW = B // _NW
_SUB = 8
_G1 = B // _SUB

_TC_ROWS = 2048
_TC_G = _TC_ROWS // _SUB

_KPAD = 102400
_MCOLS = 12800
_MROWS = _MCOLS // _SUB
_KTP = _KPAD // _SUB


def _repack_m_body(mt_ref, ex_ref, msk_ref, o_ref):
  rep = jax.lax.dot_general(
      mt_ref[...], ex_ref[...], (((0,), (0,)), ((), ())),
      preferred_element_type=jnp.float32)
  r3 = rep.reshape(_MROWS, _SUB, 128)
  z = r3 * msk_ref[...][None, :, :]
  o_ref[...] = jnp.sum(z, axis=1)


_repack_m = pl.pallas_call(
    _repack_m_body,
    grid=(_KPAD // _MCOLS,),
    in_specs=[
        pl.BlockSpec((D, _MCOLS), lambda i: (0, i)),
        pl.BlockSpec((D, 128), lambda i: (0, 0)),
        pl.BlockSpec((_SUB, 128), lambda i: (0, 0)),
    ],
    out_specs=pl.BlockSpec((_MROWS, 128), lambda i: (i, 0)),
    out_shape=jax.ShapeDtypeStruct((_KTP, 128), jnp.float32),
)


def _make_gather_kernel():
  mesh = plsc.VectorSubcoreMesh(core_axis_name="c", subcore_axis_name="s")

  @functools.partial(
      pl.kernel,
      mesh=mesh,
      compiler_params=pltpu.CompilerParams(use_tc_tiling_on_sc=False),
      out_type=jax.ShapeDtypeStruct((_G1, _SUB, 128), jnp.float32),
      scratch_types=[
          pltpu.VMEM((_BPW,), jnp.int32),
          pltpu.VMEM((_BPW, D), jnp.float32),
          pltpu.VMEM((_BPW // _SUB, _SUB, 128), jnp.float32),
          pltpu.SemaphoreType.DMA,
      ],
  )
  def k(a_hbm, m_hbm, g_hbm, idx_v, rows_v, pack_v, sem):
    wid = lax.axis_index("s") * 2 + lax.axis_index("c")
    base = wid * _BPW
    pltpu.sync_copy(a_hbm.at[pl.ds(base, _BPW)], idx_v)
    pltpu.async_copy(m_hbm.at[idx_v], rows_v, sem).wait()

    def body(t, _):
      for j in range(_SUB):
        pack_v[t, j, pl.ds(0, D)] = rows_v[t * _SUB + j, :]
      return _

    lax.fori_loop(0, _BPW // _SUB, body, 0)
    pltpu.sync_copy(pack_v, g_hbm.at[pl.ds(wid * (_BPW // _SUB), _BPW // _SUB)])

  return k


_gather = _make_gather_kernel()


def _dot_sigmoid_body(x_ref, g_ref, o_ref):
  g = g_ref[...][:, :, :D].reshape(_TC_ROWS, D)
  p = x_ref[...] * g
  z = jnp.sum(p, axis=1)
  o_ref[...] = 1.0 / (1.0 + jnp.exp(-z))


_dot_sigmoid = pl.pallas_call(
    _dot_sigmoid_body,
    grid=(B // _TC_ROWS,),
    in_specs=[
        pl.BlockSpec((_TC_ROWS, D), lambda i: (i, 0)),
        pl.BlockSpec((_TC_G, _SUB, 128), lambda i: (i, 0, 0)),
    ],
    out_specs=pl.BlockSpec((_TC_ROWS,), lambda i: (i,)),
    out_shape=jax.ShapeDtypeStruct((B,), jnp.float32),
)


@jax.jit
def kernel(X, A, m):
  ex = (lax.broadcasted_iota(jnp.int32, (D, 128), 1) % D
        == lax.broadcasted_iota(jnp.int32, (D, 128), 0)).astype(jnp.float32)
  msk = (lax.broadcasted_iota(jnp.int32, (_SUB, 128), 1) // D
         == lax.broadcasted_iota(jnp.int32, (_SUB, 128), 0)).astype(jnp.float32)
  mt_p = jnp.pad(m.T, ((0, 0), (0, _KPAD - K)))
  m5 = _repack_m(mt_p, ex, msk).reshape(_KPAD, D)
  g3 = _gather(A.astype(jnp.int32), m5)
  return _dot_sigmoid(X, g3)


# speedup vs baseline: 1.3901x; 1.0314x over previous
import functools

import jax
import jax.numpy as jnp
from jax import lax
from jax.experimental import pallas as pl
from jax.experimental.pallas import tpu as pltpu
from jax.experimental.pallas import tpu_sc as plsc

K = 100000
D = 16
B = 16384

_NW = 32
_BPW = B // _NW
_SUB = 8
_G1 = B // _SUB

_TC_ROWS = 4096
_TC_G = _TC_ROWS // _SUB

_KPAD = 102400
_MCOLS = 25600
_MROWS = _MCOLS // _SUB
_KTP = _KPAD // _SUB


def _repack_m_body(mt_ref, ex_ref, msk_ref, o_ref):
  rep = jax.lax.dot_general(
      mt_ref[...], ex_ref[...], (((0,), (0,)), ((), ())),
      preferred_element_type=jnp.float32)
  r3 = rep.reshape(_MROWS, _SUB, 128)
  z = r3 * msk_ref[...][None, :, :]
  o_ref[...] = jnp.sum(z, axis=1)


_repack_m = pl.pallas_call(
    _repack_m_body,
    grid=(_KPAD // _MCOLS,),
    in_specs=[
        pl.BlockSpec((D, _MCOLS), lambda i: (0, i)),
        pl.BlockSpec((D, 128), lambda i: (0, 0)),
        pl.BlockSpec((_SUB, 128), lambda i: (0, 0)),
    ],
    out_specs=pl.BlockSpec((_MROWS, 128), lambda i: (i, 0)),
    out_shape=jax.ShapeDtypeStruct((_KTP, 128), jnp.float32),
)


def _make_gather_kernel():
  mesh = plsc.VectorSubcoreMesh(core_axis_name="c", subcore_axis_name="s")

  @functools.partial(
      pl.kernel,
      mesh=mesh,
      compiler_params=pltpu.CompilerParams(use_tc_tiling_on_sc=False),
      out_type=jax.ShapeDtypeStruct((_G1, _SUB, 128), jnp.float32),
      scratch_types=[
          pltpu.VMEM((_BPW,), jnp.int32),
          pltpu.VMEM((_BPW, D), jnp.float32),
          pltpu.VMEM((_BPW // _SUB, _SUB, 128), jnp.float32),
          pltpu.SemaphoreType.DMA,
      ],
  )
  def k(a_hbm, m_hbm, g_hbm, idx_v, rows_v, pack_v, sem):
    wid = lax.axis_index("s") * 2 + lax.axis_index("c")
    base = wid * _BPW
    pltpu.sync_copy(a_hbm.at[pl.ds(base, _BPW)], idx_v)
    pltpu.async_copy(m_hbm.at[idx_v], rows_v, sem).wait()

    def body(t, _):
      for j in range(_SUB):
        pack_v[t, j, pl.ds(0, D)] = rows_v[t * _SUB + j, :]
      return _

    lax.fori_loop(0, _BPW // _SUB, body, 0)
    pltpu.sync_copy(pack_v, g_hbm.at[pl.ds(wid * (_BPW // _SUB), _BPW // _SUB)])

  return k


_gather = _make_gather_kernel()


def _dot_sigmoid_body(x_ref, g_ref, o_ref):
  g = g_ref[...][:, :, :D].reshape(_TC_ROWS, D)
  p = x_ref[...] * g
  z = jnp.sum(p, axis=1)
  o_ref[...] = 1.0 / (1.0 + jnp.exp(-z))


_dot_sigmoid = pl.pallas_call(
    _dot_sigmoid_body,
    grid=(B // _TC_ROWS,),
    in_specs=[
        pl.BlockSpec((_TC_ROWS, D), lambda i: (i, 0)),
        pl.BlockSpec((_TC_G, _SUB, 128), lambda i: (i, 0, 0)),
    ],
    out_specs=pl.BlockSpec((_TC_ROWS,), lambda i: (i,)),
    out_shape=jax.ShapeDtypeStruct((B,), jnp.float32),
)


@jax.jit
def kernel(X, A, m):
  ex = (lax.broadcasted_iota(jnp.int32, (D, 128), 1) % D
        == lax.broadcasted_iota(jnp.int32, (D, 128), 0)).astype(jnp.float32)
  msk = (lax.broadcasted_iota(jnp.int32, (_SUB, 128), 1) // D
         == lax.broadcasted_iota(jnp.int32, (_SUB, 128), 0)).astype(jnp.float32)
  mt_p = jnp.pad(m.T, ((0, 0), (0, _KPAD - K)))
  m5 = _repack_m(mt_p, ex, msk).reshape(_KPAD, D)
  g3 = _gather(A.astype(jnp.int32), m5)
  return _dot_sigmoid(X, g3)


# speedup vs baseline: 1.3903x; 1.0002x over previous
import functools

import jax
import jax.numpy as jnp
from jax import lax
from jax.experimental import pallas as pl
from jax.experimental.pallas import tpu as pltpu
from jax.experimental.pallas import tpu_sc as plsc

K = 100000
D = 16
B = 16384

_NW = 32
_BPW = B // _NW
_SUB = 8
_G1 = B // _SUB

_TC_ROWS = 4096
_TC_G = _TC_ROWS // _SUB

_KPAD = 102400
_MCOLS = 25600
_MROWS = _MCOLS // _SUB
_KTP = _KPAD // _SUB


def _repack_m_body(mt_ref, ex_ref, msk_ref, o_ref):
  rep = jax.lax.dot_general(
      mt_ref[...], ex_ref[...], (((0,), (0,)), ((), ())),
      preferred_element_type=jnp.float32)
  r3 = rep.reshape(_MROWS, _SUB, 128)
  z = r3 * msk_ref[...][None, :, :]
  o_ref[...] = jnp.sum(z, axis=1)


_repack_m = pl.pallas_call(
    _repack_m_body,
    grid=(_KPAD // _MCOLS,),
    in_specs=[
        pl.BlockSpec((D, _MCOLS), lambda i: (0, i)),
        pl.BlockSpec((D, 128), lambda i: (0, 0)),
        pl.BlockSpec((_SUB, 128), lambda i: (0, 0)),
    ],
    out_specs=pl.BlockSpec((_MROWS, 128), lambda i: (i, 0)),
    out_shape=jax.ShapeDtypeStruct((_KTP, 128), jnp.float32),
)


def _make_gather_kernel():
  mesh = plsc.VectorSubcoreMesh(core_axis_name="c", subcore_axis_name="s")

  @functools.partial(
      pl.kernel,
      mesh=mesh,
      compiler_params=pltpu.CompilerParams(use_tc_tiling_on_sc=False),
      out_type=jax.ShapeDtypeStruct((_G1, _SUB, 128), jnp.float32),
      scratch_types=[
          pltpu.VMEM((_BPW,), jnp.int32),
          pltpu.VMEM((_BPW // 2, D), jnp.float32),
          pltpu.VMEM((_BPW // 2, D), jnp.float32),
          pltpu.VMEM((_BPW // _SUB, _SUB, 128), jnp.float32),
          pltpu.SemaphoreType.DMA,
          pltpu.SemaphoreType.DMA,
      ],
  )
  def k(a_hbm, m_hbm, g_hbm, idx_v, rows0_v, rows1_v, pack_v, sem0, sem1):
    wid = lax.axis_index("s") * 2 + lax.axis_index("c")
    base = wid * _BPW
    half = _BPW // 2
    pltpu.sync_copy(a_hbm.at[pl.ds(base, _BPW)], idx_v)
    cp0 = pltpu.async_copy(m_hbm.at[idx_v.at[pl.ds(0, half)]], rows0_v, sem0)
    cp1 = pltpu.async_copy(m_hbm.at[idx_v.at[pl.ds(half, half)]], rows1_v,
                           sem1)

    def pack_chunk(rows_v, t0):
      def body(t, _):
        for j in range(_SUB):
          pack_v[t0 + t, j, pl.ds(0, D)] = rows_v[t * _SUB + j, :]
        return _
      lax.fori_loop(0, half // _SUB, body, 0)

    cp0.wait()
    pack_chunk(rows0_v, 0)
    cp1.wait()
    pack_chunk(rows1_v, half // _SUB)
    pltpu.sync_copy(pack_v, g_hbm.at[pl.ds(wid * (_BPW // _SUB), _BPW // _SUB)])

  return k


_gather = _make_gather_kernel()


def _dot_sigmoid_body(x_ref, g_ref, o_ref):
  g = g_ref[...][:, :, :D].reshape(_TC_ROWS, D)
  p = x_ref[...] * g
  z = jnp.sum(p, axis=1)
  o_ref[...] = 1.0 / (1.0 + jnp.exp(-z))


_dot_sigmoid = pl.pallas_call(
    _dot_sigmoid_body,
    grid=(B // _TC_ROWS,),
    in_specs=[
        pl.BlockSpec((_TC_ROWS, D), lambda i: (i, 0)),
        pl.BlockSpec((_TC_G, _SUB, 128), lambda i: (i, 0, 0)),
    ],
    out_specs=pl.BlockSpec((_TC_ROWS,), lambda i: (i,)),
    out_shape=jax.ShapeDtypeStruct((B,), jnp.float32),
)


@jax.jit
def kernel(X, A, m):
  ex = (lax.broadcasted_iota(jnp.int32, (D, 128), 1) % D
        == lax.broadcasted_iota(jnp.int32, (D, 128), 0)).astype(jnp.float32)
  msk = (lax.broadcasted_iota(jnp.int32, (_SUB, 128), 1) // D
         == lax.broadcasted_iota(jnp.int32, (_SUB, 128), 0)).astype(jnp.float32)
  mt_p = jnp.pad(m.T, ((0, 0), (0, _KPAD - K)))
  m5 = _repack_m(mt_p, ex, msk).reshape(_KPAD, D)
  g3 = _gather(A.astype(jnp.int32), m5)
  return _dot_sigmoid(X, g3)


# speedup vs baseline: 1.5759x; 1.1335x over previous
import functools

import numpy as np

import jax
import jax.numpy as jnp
from jax import lax
from jax.experimental import pallas as pl
from jax.experimental.pallas import tpu as pltpu
from jax.experimental.pallas import tpu_sc as plsc

K = 100000
D = 16
B = 16384

_NW = 32
_BPW = B // _NW
_SUB = 8
_G1 = B // _SUB

_TC_ROWS = 4096
_TC_G = _TC_ROWS // _SUB

_KPAD = 102400
_MCOLS = 25600
_MROWS = _MCOLS // _SUB
_KTP = _KPAD // _SUB


_EX = np.zeros((D, 128), np.float32)
for _d in range(D):
  _EX[_d, _d::D] = 1.0
_MSK = np.zeros((_SUB, 128), np.float32)
for _s in range(_SUB):
  _MSK[_s, _s * D:(_s + 1) * D] = 1.0


def _repack_m_body(mt_ref, ex_ref, msk_ref, o_ref):
  rep = jax.lax.dot_general(
      mt_ref[...], ex_ref[...], (((0,), (0,)), ((), ())),
      preferred_element_type=jnp.float32)
  r3 = rep.reshape(_MROWS, _SUB, 128)
  z = r3 * msk_ref[...][None, :, :]
  o_ref[...] = jnp.sum(z, axis=1)


_repack_m = pl.pallas_call(
    _repack_m_body,
    grid=(_KPAD // _MCOLS,),
    in_specs=[
        pl.BlockSpec((D, _MCOLS), lambda i: (0, i)),
        pl.BlockSpec((D, 128), lambda i: (0, 0)),
        pl.BlockSpec((_SUB, 128), lambda i: (0, 0)),
    ],
    out_specs=pl.BlockSpec((_MROWS, 128), lambda i: (i, 0)),
    out_shape=jax.ShapeDtypeStruct((_KTP, 128), jnp.float32),
)


def _make_gather_kernel():
  mesh = plsc.VectorSubcoreMesh(core_axis_name="c", subcore_axis_name="s")

  @functools.partial(
      pl.kernel,
      mesh=mesh,
      compiler_params=pltpu.CompilerParams(use_tc_tiling_on_sc=False),
      out_type=jax.ShapeDtypeStruct((_G1, _SUB, 128), jnp.float32),
      scratch_types=[
          pltpu.VMEM((_BPW,), jnp.int32),
          pltpu.VMEM((_BPW // 2, D), jnp.float32),
          pltpu.VMEM((_BPW // 2, D), jnp.float32),
          pltpu.VMEM((_BPW // _SUB, _SUB, 128), jnp.float32),
          pltpu.SemaphoreType.DMA,
          pltpu.SemaphoreType.DMA,
      ],
  )
  def k(a_hbm, m_hbm, g_hbm, idx_v, rows0_v, rows1_v, pack_v, sem0, sem1):
    wid = lax.axis_index("s") * 2 + lax.axis_index("c")
    base = wid * _BPW
    half = _BPW // 2
    pltpu.sync_copy(a_hbm.at[pl.ds(base, _BPW)], idx_v)
    cp0 = pltpu.async_copy(m_hbm.at[idx_v.at[pl.ds(0, half)]], rows0_v, sem0)
    cp1 = pltpu.async_copy(m_hbm.at[idx_v.at[pl.ds(half, half)]], rows1_v,
                           sem1)

    def pack_chunk(rows_v, t0):
      def body(t, _):
        for j in range(_SUB):
          pack_v[t0 + t, j, pl.ds(0, D)] = rows_v[t * _SUB + j, :]
        return _
      lax.fori_loop(0, half // _SUB, body, 0)

    cp0.wait()
    pack_chunk(rows0_v, 0)
    cp1.wait()
    pack_chunk(rows1_v, half // _SUB)
    pltpu.sync_copy(pack_v, g_hbm.at[pl.ds(wid * (_BPW // _SUB), _BPW // _SUB)])

  return k


_gather = _make_gather_kernel()


def _dot_sigmoid_body(x_ref, g_ref, o_ref):
  g = g_ref[...][:, :, :D].reshape(_TC_ROWS, D)
  p = x_ref[...] * g
  z = jnp.sum(p, axis=1)
  o_ref[...] = 1.0 / (1.0 + jnp.exp(-z))


_dot_sigmoid = pl.pallas_call(
    _dot_sigmoid_body,
    grid=(B // _TC_ROWS,),
    in_specs=[
        pl.BlockSpec((_TC_ROWS, D), lambda i: (i, 0)),
        pl.BlockSpec((_TC_G, _SUB, 128), lambda i: (i, 0, 0)),
    ],
    out_specs=pl.BlockSpec((_TC_ROWS,), lambda i: (i,)),
    out_shape=jax.ShapeDtypeStruct((B,), jnp.float32),
)


@jax.jit
def kernel(X, A, m):
  m5 = _repack_m(m.T, jnp.asarray(_EX), jnp.asarray(_MSK)).reshape(_KPAD, D)
  g3 = _gather(A.astype(jnp.int32), m5)
  return _dot_sigmoid(X, g3)
